# Initial kernel scaffold; baseline (speedup 1.0000x reference)
import jax
import jax.numpy as jnp
from jax.experimental import pallas as pl


def kernel(x, edge_index, metal_id, pred_pos, metal_table, mp_W, mp_b, g1_W, g1_b, g2_W, g2_b, lp_W, lp_b, gt_W1, gt_b1, gt_W2, gt_b2, pr_W1, pr_b1, pr_W2, pr_b2):
    raise NotImplementedError("write your pallas kernel here")


# speedup vs baseline: 8.8231x; 8.8231x over previous
import functools

import jax
import jax.numpy as jnp
from jax import lax
from jax.experimental import pallas as pl
from jax.experimental.pallas import tpu as pltpu
from jax.experimental.pallas import tpu_sc as plsc

N = 10000
E = 320000
D = 128
H = 128
NC = 2
NS = 16
NW = NC * NS
L = 16

NPAD = 10240
SLOP = N
CH = 128
EW = 10112
EPAD = NW * EW
NCHUNK = EW // CH
RSTRIPE = NPAD // NS
BR = 1024
GRID = NPAD // BR

@functools.cache
def _mesh():
  return plsc.VectorSubcoreMesh(
      core_axis_name="c", subcore_axis_name="s", num_cores=NC,
      num_subcores=NS)


def _zero_vmem_1d(ref, n):
  def body(i, _):
    ref[pl.ds(i * L, L)] = jnp.zeros((L,), jnp.float32)
    return 0
  lax.fori_loop(0, n // L, body, 0, unroll=4)


def _zero_vmem_2d(ref, rows, cols):
  def body(i, _):
    r = i // (cols // L)
    k = i % (cols // L)
    ref[r, pl.ds(k * L, L)] = jnp.zeros((L,), jnp.float32)
    return 0
  lax.fori_loop(0, rows * (cols // L), body, 0, unroll=4)


@functools.cache
def _sc_deg_kernel():
  return functools.partial(
      pl.kernel,
      out_type=jax.ShapeDtypeStruct((NW, NPAD), jnp.float32),
      mesh=_mesh(),
      compiler_params=pltpu.CompilerParams(needs_layout_passes=False),
      scratch_types=[
          pltpu.VMEM((1, CH), jnp.int32),
          pltpu.VMEM((NPAD,), jnp.float32),
      ],
  )(_sc_deg_body)


def _sc_deg_body(dst_hbm, out_hbm, idx_v, deg_v):
  c = lax.axis_index("c")
  s = lax.axis_index("s")
  wid = c * NS + s
  _zero_vmem_1d(deg_v, NPAD)
  base = wid * EW

  def chunk(j, _):
    pltpu.sync_copy(dst_hbm.at[pl.ds(base + j * CH, CH)], idx_v.at[0])

    def vreg(k, _):
      d16 = idx_v[0, pl.ds(k * L, L)]
      cnt, last = plsc.scan_count(d16)
      plsc.addupdate_scatter(deg_v, [d16], cnt.astype(jnp.float32), mask=last)
      return 0

    lax.fori_loop(0, CH // L, vreg, 0, unroll=8)
    return 0

  lax.fori_loop(0, NCHUNK, chunk, 0)
  pltpu.sync_copy(deg_v, out_hbm.at[wid])


@functools.cache
def _sc_agg_kernel():
  return functools.partial(
      pl.kernel,
      out_type=jax.ShapeDtypeStruct((NC, NPAD, H), jnp.float32),
      mesh=_mesh(),
      scratch_types=[
          pltpu.VMEM((1, CH), jnp.int32),
          pltpu.VMEM((1, CH), jnp.int32),
          pltpu.VMEM((CH, H), jnp.float32),
          pltpu.VMEM((CH, H), jnp.float32),
          pltpu.VMEM_SHARED((NPAD, H), jnp.float32),
          pltpu.SemaphoreType.DMA,
      ],
  )(_sc_agg_body)


def _sc_agg_body(src_hbm, dst_hbm, g_hbm, out_hbm, sidx, didx, rows, zrows,
                 s_sh, sem):
  c = lax.axis_index("c")
  s = lax.axis_index("s")
  wid = c * NS + s
  _zero_vmem_2d(zrows, CH, H)
  for k in range(RSTRIPE // CH):
    pltpu.sync_copy(zrows, s_sh.at[pl.ds(s * RSTRIPE + k * CH, CH)])
  plsc.subcore_barrier()
  base = wid * EW

  def chunk(j, _):
    b = base + j * CH
    pltpu.sync_copy(src_hbm.at[pl.ds(b, CH)], sidx.at[0])
    pltpu.sync_copy(dst_hbm.at[pl.ds(b, CH)], didx.at[0])
    pltpu.async_copy(g_hbm.at[sidx.at[0]], rows, sem).wait()
    pltpu.sync_copy(rows, s_sh.at[didx.at[0]], add=True)
    return 0

  lax.fori_loop(0, NCHUNK, chunk, 0)
  plsc.subcore_barrier()
  for k in range(RSTRIPE // CH):
    r0 = s * RSTRIPE + k * CH
    pltpu.sync_copy(s_sh.at[pl.ds(r0, CH)], out_hbm.at[c].at[pl.ds(r0, CH)])


def _sc_deg(dstp):
  return _sc_deg_kernel()(dstp)


def _sc_agg(srcp, dstp, g):
  return _sc_agg_kernel()(srcp, dstp, g)


def _tc_norm_g1_body(deg_ref, x_ref, w_ref, norm_ref, g1_ref):
  degsum = lax.dot_general(
      deg_ref[...], jnp.ones((NW, 1), jnp.float32),
      (((0,), (0,)), ((), ())), preferred_element_type=jnp.float32)
  norm = lax.rsqrt(degsum + 1.0)
  norm_ref[...] = norm
  h = jnp.dot(x_ref[...], w_ref[...], preferred_element_type=jnp.float32)
  g1_ref[...] = h * norm


def _tc_norm_g1(degp, xp, w1):
  return pl.pallas_call(
      _tc_norm_g1_body,
      grid=(GRID,),
      in_specs=[
          pl.BlockSpec((NW, BR), lambda i: (0, i)),
          pl.BlockSpec((BR, D), lambda i: (i, 0)),
          pl.BlockSpec((D, H), lambda i: (0, 0)),
      ],
      out_specs=[
          pl.BlockSpec((BR, 1), lambda i: (i, 0)),
          pl.BlockSpec((BR, H), lambda i: (i, 0)),
      ],
      out_shape=[
          jax.ShapeDtypeStruct((NPAD, 1), jnp.float32),
          jax.ShapeDtypeStruct((NPAD, H), jnp.float32),
      ],
  )(degp, xp, w1)


def _tc_h1_g2_body(sp_ref, g1_ref, norm_ref, b1_ref, w2_ref, g2_ref):
  ssum = sp_ref[0] + sp_ref[1]
  norm = norm_ref[...]
  h1 = jnp.maximum(norm * (ssum + g1_ref[...]) + b1_ref[...], 0.0)
  h = jnp.dot(h1, w2_ref[...], preferred_element_type=jnp.float32)
  g2_ref[...] = h * norm


def _tc_h1_g2(s1p, g1, normc, b1r, w2):
  return pl.pallas_call(
      _tc_h1_g2_body,
      grid=(GRID,),
      in_specs=[
          pl.BlockSpec((NC, BR, H), lambda i: (0, i, 0)),
          pl.BlockSpec((BR, H), lambda i: (i, 0)),
          pl.BlockSpec((BR, 1), lambda i: (i, 0)),
          pl.BlockSpec((1, H), lambda i: (0, 0)),
          pl.BlockSpec((H, H), lambda i: (0, 0)),
      ],
      out_specs=pl.BlockSpec((BR, H), lambda i: (i, 0)),
      out_shape=jax.ShapeDtypeStruct((NPAD, H), jnp.float32),
  )(s1p, g1, normc, b1r, w2)


def _tc_head_body(s2p_ref, g2_ref, norm_ref, pred_ref, mid_ref, mtab_ref,
                  mpw_ref, mpb_ref, b2_ref, lpw_ref, lpb_ref, gtw1_ref,
                  gtb1_ref, gtw2_ref, gtb2_ref, prw1_ref, prb1_ref, prw2_ref,
                  prb2_ref, out_ref, acc_s, acc_g, acc_n):
  i = pl.program_id(0)

  @pl.when(i == 0)
  def _():
    acc_s[...] = jnp.zeros_like(acc_s)
    acc_g[...] = jnp.zeros_like(acc_g)
    acc_n[...] = jnp.zeros_like(acc_n)

  rowids = lax.broadcasted_iota(jnp.int32, (BR, 8), 0) + i * BR
  onehot = (rowids == pred_ref[...]).astype(jnp.float32)
  ssum = s2p_ref[0] + s2p_ref[1]
  acc_s[...] += lax.dot_general(onehot, ssum, (((0,), (0,)), ((), ())),
                                preferred_element_type=jnp.float32)
  acc_g[...] += lax.dot_general(onehot, g2_ref[...], (((0,), (0,)), ((), ())),
                                preferred_element_type=jnp.float32)
  acc_n[...] += lax.dot_general(onehot, norm_ref[...],
                                (((0,), (0,)), ((), ())),
                                preferred_element_type=jnp.float32)

  @pl.when(i == GRID - 1)
  def _():
    h2 = jnp.maximum(acc_n[...] * (acc_s[...] + acc_g[...]) + b2_ref[...],
                     0.0)
    h_b = jnp.maximum(
        jnp.dot(h2, lpw_ref[...], preferred_element_type=jnp.float32)
        + lpb_ref[...], 0.0)
    t = jnp.tanh(
        jnp.dot(h_b, gtw1_ref[...], preferred_element_type=jnp.float32)
        + gtb1_ref[...])
    gate = jax.nn.sigmoid(
        jnp.dot(t, gtw2_ref[...], preferred_element_type=jnp.float32)
        + gtb2_ref[...])
    msel = (lax.broadcasted_iota(jnp.int32, (1, 32), 1)
            == mid_ref[...]).astype(jnp.float32)
    memb = jnp.dot(msel, mtab_ref[...], preferred_element_type=jnp.float32)
    mfeat = jnp.maximum(
        jnp.dot(memb, mpw_ref[...], preferred_element_type=jnp.float32)
        + mpb_ref[...], 0.0)
    comb = gate * h_b + mfeat
    p1 = jnp.dot(comb, prw1_ref[...], preferred_element_type=jnp.float32)
    p1 = p1 + prb1_ref[...]
    out_ref[...] = (jnp.dot(p1, prw2_ref[...],
                            preferred_element_type=jnp.float32)
                    + prb2_ref[...])


def _tc_head(s2p, g2, normc, predr, midr, mtab, mpw, mpbr, b2r, lpw, lpbr,
             gtw1, gtb1r, gtw2, gtb2r, prw1, prb1r, prw2, prb2r):
  full = lambda shape: pl.BlockSpec(shape, lambda i: tuple(0 for _ in shape))
  return pl.pallas_call(
      _tc_head_body,
      grid=(GRID,),
      in_specs=[
          pl.BlockSpec((NC, BR, H), lambda i: (0, i, 0)),
          pl.BlockSpec((BR, H), lambda i: (i, 0)),
          pl.BlockSpec((BR, 1), lambda i: (i, 0)),
          full((1, 8)),
          full((1, 1)),
          full((32, 64)),
          full((64, H)),
          full((1, H)),
          full((1, H)),
          full((H, H)),
          full((1, H)),
          full((H, H // 2)),
          full((1, H // 2)),
          full((H // 2, 1)),
          full((1, 1)),
          full((H, H // 2)),
          full((1, H // 2)),
          full((H // 2, 1)),
          full((1, 1)),
      ],
      out_specs=pl.BlockSpec((8, 1), lambda i: (0, 0)),
      out_shape=jax.ShapeDtypeStruct((8, 1), jnp.float32),
      scratch_shapes=[
          pltpu.VMEM((8, H), jnp.float32),
          pltpu.VMEM((8, H), jnp.float32),
          pltpu.VMEM((8, 1), jnp.float32),
      ],
  )(s2p, g2, normc, predr, midr, mtab, mpw, mpbr, b2r, lpw, lpbr, gtw1,
    gtb1r, gtw2, gtb2r, prw1, prb1r, prw2, prb2r)


def kernel(x, edge_index, metal_id, pred_pos, metal_table, mp_W, mp_b, g1_W,
           g1_b, g2_W, g2_b, lp_W, lp_b, gt_W1, gt_b1, gt_W2, gt_b2, pr_W1,
           pr_b1, pr_W2, pr_b2):
  src = edge_index[0]
  dst = edge_index[1]
  srcp = jnp.concatenate(
      [src, jnp.zeros((EPAD - E,), jnp.int32)])
  dstp = jnp.concatenate(
      [dst, jnp.full((EPAD - E,), SLOP, jnp.int32)])
  xp = jnp.pad(x, ((0, NPAD - N), (0, 0)))

  degp = _sc_deg(dstp)
  normc, g1 = _tc_norm_g1(degp, xp, g1_W)
  s1p = _sc_agg(srcp, dstp, g1)
  g2 = _tc_h1_g2(s1p, g1, normc, g1_b.reshape(1, H), g2_W)
  s2p = _sc_agg(srcp, dstp, g2)
  out = _tc_head(
      s2p, g2, normc, pred_pos.reshape(1, 8), metal_id.reshape(1, 1),
      metal_table, mp_W, mp_b.reshape(1, H), g2_b.reshape(1, H), lp_W,
      lp_b.reshape(1, H), gt_W1, gt_b1.reshape(1, H // 2), gt_W2,
      gt_b2.reshape(1, 1), pr_W1, pr_b1.reshape(1, H // 2), pr_W2,
      pr_b2.reshape(1, 1))
  return out
